# Initial kernel scaffold; baseline (speedup 1.0000x reference)
import jax
import jax.numpy as jnp
from jax.experimental import pallas as pl


def kernel(input_seq, seq_lengths, supports, W_g0, b_g0, W_c0, b_c0, W_g1, b_g1, W_c1, b_c1, W_fc, b_fc, W_id, b_id):
    raise NotImplementedError("write your pallas kernel here")


# speedup vs baseline: 11.1401x; 11.1401x over previous
import jax
import jax.numpy as jnp
from jax.experimental import pallas as pl
from jax.experimental.pallas import tpu as pltpu

N = 128
F = 64
H = 64
B = 32
T = 60
NMAT = 5
ZW = F + H
C_SEIZ = 4
C_ID = 100


def _dot(a, b):
    return jnp.dot(a, b, preferred_element_type=jnp.float32)


def _dcrnn_body(seq_ref, idx_ref, sup_ref,
                wg0_ref, bg0_ref, wc0_ref, bc0_ref,
                wg1_ref, bg1_ref, wc1_ref, bc1_ref,
                wfc_ref, bfc_ref, wid_ref, bid_ref,
                out_seiz_ref, out_id_ref,
                z1, z2, rst_s, last):
    t = pl.program_id(0)

    @pl.when(t == 0)
    def _init():
        z1[:] = jnp.zeros((N, B * ZW), jnp.float32)
        z2[:] = jnp.zeros((N, B * ZW), jnp.float32)
        last[:] = jnp.zeros((B, N, H), jnp.float32)

    for b in range(B):
        z1[:, b * ZW:b * ZW + F] = seq_ref[0, b]

    s0 = sup_ref[0]
    s1 = sup_ref[1]

    def layer(z_ref, wg_ref, bg_ref, wc_ref, bc_ref, sink):
        z = z_ref[:]
        a1 = _dot(s0, z)
        a2 = 2.0 * _dot(s0, a1) - z
        e1 = _dot(s1, z)
        e2 = 2.0 * _dot(s1, e1) - z
        dz = [z, a1, a2, e1, e2]
        us = []
        for b in range(B):
            lo = b * ZW
            xcat = jnp.concatenate([d[:, lo:lo + ZW] for d in dz], axis=1)
            g = jax.nn.sigmoid(_dot(xcat, wg_ref[:]) + bg_ref[:])
            st_b = z[:, lo + F:lo + ZW]
            rst_s[:, b * H:(b + 1) * H] = g[:, :H] * st_b
            us.append(g[:, H:])
        rst = rst_s[:]
        c1 = _dot(s0, rst)
        c2 = 2.0 * _dot(s0, c1) - rst
        d1 = _dot(s1, rst)
        d2 = 2.0 * _dot(s1, d1) - rst
        drst = [rst, c1, c2, d1, d2]
        for b in range(B):
            lo = b * ZW
            parts = []
            for m in range(NMAT):
                parts.append(dz[m][:, lo:lo + F])
                parts.append(drst[m][:, b * H:(b + 1) * H])
            ccat = jnp.concatenate(parts, axis=1)
            c = jnp.tanh(_dot(ccat, wc_ref[:]) + bc_ref[:])
            u = us[b]
            st_b = z[:, lo + F:lo + ZW]
            new_b = u * st_b + (1.0 - u) * c
            z_ref[:, lo + F:lo + ZW] = new_b
            sink(b, new_b)

    def sink1(b, new_b):
        z2[:, b * ZW:b * ZW + F] = new_b

    def sink2(b, new_b):
        @pl.when(idx_ref[b] == t)
        def _store():
            last[b] = new_b

    layer(z1, wg0_ref, bg0_ref, wc0_ref, bc0_ref, sink1)
    layer(z2, wg1_ref, bg1_ref, wc1_ref, bc1_ref, sink2)

    @pl.when(t == T - 1)
    def _heads():
        h = jnp.maximum(last[:], 0.0).reshape(B * N, H)
        lg = _dot(h, wfc_ref[:]) + bfc_ref[:]
        out_seiz_ref[:] = jnp.max(lg.reshape(B, N, C_SEIZ), axis=1)
        li = _dot(h, wid_ref[:]) + bid_ref[:]
        out_id_ref[:] = jnp.max(li.reshape(B, N, C_ID), axis=1)


def kernel(input_seq, seq_lengths, supports, W_g0, b_g0, W_c0, b_c0,
           W_g1, b_g1, W_c1, b_c1, W_fc, b_fc, W_id, b_id):
    seq = jnp.transpose(input_seq, (1, 0, 2, 3))
    idx = jnp.clip(seq_lengths.astype(jnp.int32) - 1, 0, T - 1)

    def regroup(w, in_size):
        return w.reshape(in_size, NMAT, -1).transpose(1, 0, 2).reshape(
            NMAT * in_size, -1)

    wg0 = regroup(W_g0, ZW)
    wc0 = regroup(W_c0, ZW)
    wg1 = regroup(W_g1, ZW)
    wc1 = regroup(W_c1, ZW)

    full = lambda a: pl.BlockSpec(a.shape, lambda t: (0,) * a.ndim)
    args = (seq, idx, supports,
            wg0, b_g0.reshape(1, -1), wc0, b_c0.reshape(1, -1),
            wg1, b_g1.reshape(1, -1), wc1, b_c1.reshape(1, -1),
            W_fc, b_fc.reshape(1, -1), W_id, b_id.reshape(1, -1))
    in_specs = [
        pl.BlockSpec((1, B, N, F), lambda t: (t, 0, 0, 0)),
        pl.BlockSpec(memory_space=pltpu.SMEM),
    ] + [full(a) for a in args[2:]]

    out_seiz, out_id = pl.pallas_call(
        _dcrnn_body,
        grid=(T,),
        in_specs=in_specs,
        out_specs=[
            pl.BlockSpec((B, C_SEIZ), lambda t: (0, 0)),
            pl.BlockSpec((B, C_ID), lambda t: (0, 0)),
        ],
        out_shape=[
            jax.ShapeDtypeStruct((B, C_SEIZ), jnp.float32),
            jax.ShapeDtypeStruct((B, C_ID), jnp.float32),
        ],
        scratch_shapes=[
            pltpu.VMEM((N, B * ZW), jnp.float32),
            pltpu.VMEM((N, B * ZW), jnp.float32),
            pltpu.VMEM((N, B * H), jnp.float32),
            pltpu.VMEM((B, N, H), jnp.float32),
        ],
        compiler_params=pltpu.CompilerParams(
            dimension_semantics=("arbitrary",),
        ),
    )(*args)
    return (out_seiz, out_id)


# speedup vs baseline: 15.9438x; 1.4312x over previous
import jax
import jax.numpy as jnp
from jax.experimental import pallas as pl
from jax.experimental.pallas import tpu as pltpu

N = 128
F = 64
H = 64
B = 32
T = 60
NMAT = 5
C_SEIZ = 4
C_ID = 100


def _dot(a, b):
    return jnp.dot(a, b, preferred_element_type=jnp.float32)


def _dcrnn_body(seq_ref, idx_ref, sup_ref,
                wg0_ref, bg0_ref, wc0_ref, bc0_ref,
                wg1_ref, bg1_ref, wc1_ref, bc1_ref,
                wfc_ref, bfc_ref, wid_ref, bid_ref,
                out_seiz_ref, out_id_ref,
                st1, st2, last):
    t = pl.program_id(0)

    @pl.when(t == 0)
    def _init():
        st1[:] = jnp.zeros((N, B * H), jnp.float32)
        st2[:] = jnp.zeros((N, B * H), jnp.float32)
        last[:] = jnp.zeros((B, N, H), jnp.float32)

    s0 = sup_ref[0]
    s1 = sup_ref[1]

    def diffuse(x):
        a1 = _dot(s0, x)
        a2 = 2.0 * _dot(s0, a1) - x
        e1 = _dot(s1, x)
        e2 = 2.0 * _dot(s1, e1) - x
        return [x, a1, a2, e1, e2]

    def layer(xs, st_ref, wg_ref, bg_ref, wc_ref, bc_ref):
        st = st_ref[:]
        comb = jnp.concatenate(xs + [st], axis=1)
        dc = diffuse(comb)
        us = []
        rs = []
        for b in range(B):
            parts = []
            for m in range(NMAT):
                parts.append(dc[m][:, b * F:(b + 1) * F])
                parts.append(dc[m][:, B * F + b * H:B * F + (b + 1) * H])
            xcat = jnp.concatenate(parts, axis=1)
            g = jax.nn.sigmoid(_dot(xcat, wg_ref[:]) + bg_ref[:])
            rs.append(g[:, :H] * st[:, b * H:(b + 1) * H])
            us.append(g[:, H:])
        drst = diffuse(jnp.concatenate(rs, axis=1))
        outs = []
        for b in range(B):
            parts = []
            for m in range(NMAT):
                parts.append(dc[m][:, b * F:(b + 1) * F])
                parts.append(drst[m][:, b * H:(b + 1) * H])
            ccat = jnp.concatenate(parts, axis=1)
            c = jnp.tanh(_dot(ccat, wc_ref[:]) + bc_ref[:])
            st_b = st[:, b * H:(b + 1) * H]
            outs.append(us[b] * st_b + (1.0 - us[b]) * c)
        for b in range(B):
            st_ref[:, b * H:(b + 1) * H] = outs[b]
        return outs

    xs = [seq_ref[0, b] for b in range(B)]
    mid = layer(xs, st1, wg0_ref, bg0_ref, wc0_ref, bc0_ref)
    fin = layer(mid, st2, wg1_ref, bg1_ref, wc1_ref, bc1_ref)

    for b in range(B):
        @pl.when(idx_ref[b] == t)
        def _store(b=b):
            last[b] = fin[b]

    @pl.when(t == T - 1)
    def _heads():
        h = jnp.maximum(last[:], 0.0).reshape(B * N, H)
        lg = _dot(h, wfc_ref[:]) + bfc_ref[:]
        out_seiz_ref[:] = jnp.max(lg.reshape(B, N, C_SEIZ), axis=1)
        li = _dot(h, wid_ref[:]) + bid_ref[:]
        out_id_ref[:] = jnp.max(li.reshape(B, N, C_ID), axis=1)


def kernel(input_seq, seq_lengths, supports, W_g0, b_g0, W_c0, b_c0,
           W_g1, b_g1, W_c1, b_c1, W_fc, b_fc, W_id, b_id):
    seq = jnp.transpose(input_seq, (1, 0, 2, 3))
    idx = jnp.clip(seq_lengths.astype(jnp.int32) - 1, 0, T - 1)

    def regroup(w, in_size):
        return w.reshape(in_size, NMAT, -1).transpose(1, 0, 2).reshape(
            NMAT * in_size, -1)

    wg0 = regroup(W_g0, F + H)
    wc0 = regroup(W_c0, F + H)
    wg1 = regroup(W_g1, F + H)
    wc1 = regroup(W_c1, F + H)

    full = lambda a: pl.BlockSpec(a.shape, lambda t: (0,) * a.ndim)
    args = (seq, idx, supports,
            wg0, b_g0.reshape(1, -1), wc0, b_c0.reshape(1, -1),
            wg1, b_g1.reshape(1, -1), wc1, b_c1.reshape(1, -1),
            W_fc, b_fc.reshape(1, -1), W_id, b_id.reshape(1, -1))
    in_specs = [
        pl.BlockSpec((1, B, N, F), lambda t: (t, 0, 0, 0)),
        pl.BlockSpec(memory_space=pltpu.SMEM),
    ] + [full(a) for a in args[2:]]

    out_seiz, out_id = pl.pallas_call(
        _dcrnn_body,
        grid=(T,),
        in_specs=in_specs,
        out_specs=[
            pl.BlockSpec((B, C_SEIZ), lambda t: (0, 0)),
            pl.BlockSpec((B, C_ID), lambda t: (0, 0)),
        ],
        out_shape=[
            jax.ShapeDtypeStruct((B, C_SEIZ), jnp.float32),
            jax.ShapeDtypeStruct((B, C_ID), jnp.float32),
        ],
        scratch_shapes=[
            pltpu.VMEM((N, B * H), jnp.float32),
            pltpu.VMEM((N, B * H), jnp.float32),
            pltpu.VMEM((B, N, H), jnp.float32),
        ],
        compiler_params=pltpu.CompilerParams(
            dimension_semantics=("arbitrary",),
        ),
    )(*args)
    return (out_seiz, out_id)


# speedup vs baseline: 17.9692x; 1.1270x over previous
import jax
import jax.numpy as jnp
from jax.experimental import pallas as pl
from jax.experimental.pallas import tpu as pltpu

N = 128
F = 64
H = 64
B = 32
T = 60
NMAT = 5
C_SEIZ = 4
C_ID = 100


def _dot(a, b):
    return jnp.dot(a, b, preferred_element_type=jnp.float32)


def _bf(x):
    return x.astype(jnp.bfloat16)


def _dcrnn_body(seq_ref, idx_ref, sup_ref,
                wg0_ref, bg0_ref, wc0_ref, bc0_ref,
                wg1_ref, bg1_ref, wc1_ref, bc1_ref,
                wfc_ref, bfc_ref, wid_ref, bid_ref,
                out_seiz_ref, out_id_ref,
                st1, st2, last):
    t = pl.program_id(0)

    @pl.when(t == 0)
    def _init():
        st1[:] = jnp.zeros((N, B * H), jnp.float32)
        st2[:] = jnp.zeros((N, B * H), jnp.float32)
        last[:] = jnp.zeros((B, N, H), jnp.float32)

    s0 = sup_ref[0]
    s1 = sup_ref[1]

    def diffuse(xb):
        a1 = _bf(_dot(s0, xb))
        a2 = _bf(2.0 * _dot(s0, a1).astype(jnp.float32) - xb.astype(jnp.float32))
        e1 = _bf(_dot(s1, xb))
        e2 = _bf(2.0 * _dot(s1, e1).astype(jnp.float32) - xb.astype(jnp.float32))
        return [xb, a1, a2, e1, e2]

    def layer(xs, st_ref, wg_ref, bg_ref, wc_ref, bc_ref):
        st = st_ref[:]
        comb = jnp.concatenate(xs + [_bf(st)], axis=1)
        dc = diffuse(comb)
        us = []
        rs = []
        for b in range(B):
            parts = []
            for m in range(NMAT):
                parts.append(dc[m][:, b * F:(b + 1) * F])
                parts.append(dc[m][:, B * F + b * H:B * F + (b + 1) * H])
            xcat = jnp.concatenate(parts, axis=1)
            g = jax.nn.sigmoid(_dot(xcat, wg_ref[:]) + bg_ref[:])
            rs.append(g[:, :H] * st[:, b * H:(b + 1) * H])
            us.append(g[:, H:])
        drst = diffuse(_bf(jnp.concatenate(rs, axis=1)))
        outs = []
        for b in range(B):
            parts = []
            for m in range(NMAT):
                parts.append(dc[m][:, b * F:(b + 1) * F])
                parts.append(drst[m][:, b * H:(b + 1) * H])
            ccat = jnp.concatenate(parts, axis=1)
            c = jnp.tanh(_dot(ccat, wc_ref[:]) + bc_ref[:])
            st_b = st[:, b * H:(b + 1) * H]
            outs.append(us[b] * st_b + (1.0 - us[b]) * c)
        for b in range(B):
            st_ref[:, b * H:(b + 1) * H] = outs[b]
        return outs

    xs = [seq_ref[0, b] for b in range(B)]
    mid = layer(xs, st1, wg0_ref, bg0_ref, wc0_ref, bc0_ref)
    fin = layer([_bf(o) for o in mid], st2, wg1_ref, bg1_ref, wc1_ref, bc1_ref)

    for b in range(B):
        @pl.when(idx_ref[b] == t)
        def _store(b=b):
            last[b] = fin[b]

    @pl.when(t == T - 1)
    def _heads():
        h = jnp.maximum(last[:], 0.0).reshape(B * N, H)
        lg = _dot(h, wfc_ref[:]) + bfc_ref[:]
        out_seiz_ref[:] = jnp.max(lg.reshape(B, N, C_SEIZ), axis=1)
        li = _dot(h, wid_ref[:]) + bid_ref[:]
        out_id_ref[:] = jnp.max(li.reshape(B, N, C_ID), axis=1)


def kernel(input_seq, seq_lengths, supports, W_g0, b_g0, W_c0, b_c0,
           W_g1, b_g1, W_c1, b_c1, W_fc, b_fc, W_id, b_id):
    seq = jnp.transpose(input_seq, (1, 0, 2, 3)).astype(jnp.bfloat16)
    idx = jnp.clip(seq_lengths.astype(jnp.int32) - 1, 0, T - 1)
    supports = supports.astype(jnp.bfloat16)

    def regroup(w, in_size):
        return w.reshape(in_size, NMAT, -1).transpose(1, 0, 2).reshape(
            NMAT * in_size, -1).astype(jnp.bfloat16)

    wg0 = regroup(W_g0, F + H)
    wc0 = regroup(W_c0, F + H)
    wg1 = regroup(W_g1, F + H)
    wc1 = regroup(W_c1, F + H)

    full = lambda a: pl.BlockSpec(a.shape, lambda t: (0,) * a.ndim)
    args = (seq, idx, supports,
            wg0, b_g0.reshape(1, -1), wc0, b_c0.reshape(1, -1),
            wg1, b_g1.reshape(1, -1), wc1, b_c1.reshape(1, -1),
            W_fc, b_fc.reshape(1, -1), W_id, b_id.reshape(1, -1))
    in_specs = [
        pl.BlockSpec((1, B, N, F), lambda t: (t, 0, 0, 0)),
        pl.BlockSpec(memory_space=pltpu.SMEM),
    ] + [full(a) for a in args[2:]]

    out_seiz, out_id = pl.pallas_call(
        _dcrnn_body,
        grid=(T,),
        in_specs=in_specs,
        out_specs=[
            pl.BlockSpec((B, C_SEIZ), lambda t: (0, 0)),
            pl.BlockSpec((B, C_ID), lambda t: (0, 0)),
        ],
        out_shape=[
            jax.ShapeDtypeStruct((B, C_SEIZ), jnp.float32),
            jax.ShapeDtypeStruct((B, C_ID), jnp.float32),
        ],
        scratch_shapes=[
            pltpu.VMEM((N, B * H), jnp.float32),
            pltpu.VMEM((N, B * H), jnp.float32),
            pltpu.VMEM((B, N, H), jnp.float32),
        ],
        compiler_params=pltpu.CompilerParams(
            dimension_semantics=("arbitrary",),
        ),
    )(*args)
    return (out_seiz, out_id)
